# baseline (speedup 1.0000x reference)
import functools

import jax
import jax.numpy as jnp
from jax import lax
from jax.experimental import pallas as pl
from jax.experimental.pallas import tpu as pltpu
from jax.experimental.pallas import tpu_sc as plsc

NC = 2
NS = 16
NW = NC * NS
L = 128
LANES = 16
HW = 16
NACC = 10240
ROWS_PER = NACC // NS

CH0 = 116
CH1 = 44
CHU = 80


def _mesh():
    return plsc.VectorSubcoreMesh(
        core_axis_name="c", subcore_axis_name="s", num_cores=NC, num_subcores=NS
    )


def _fill(buf, val):
    f = buf.shape[1]

    def row(r, c):
        for cc in range(f // LANES):
            buf[r, pl.ds(LANES * cc, LANES)] = jnp.full((LANES,), val, jnp.float32)
        return c

    lax.fori_loop(0, buf.shape[0], row, 0)


def _zero_acc(src_v, acc_sh, sid):
    for r5 in range(ROWS_PER // L):
        pltpu.sync_copy(src_v, acc_sh.at[pl.ds(sid * ROWS_PER + L * r5, L)])


def _unpack_chunk(packed_v, j, idxbuf, base, want_src):
    for r in range(L // LANES):
        pv = packed_v[j, pl.ds(LANES * r, LANES)]
        if want_src:
            idxbuf[base, pl.ds(LANES * r, LANES)] = lax.shift_right_logical(pv, 16)
        idxbuf[base + 1, pl.ds(LANES * r, LANES)] = pv & 0xFFFF


def _make_hist(f):
    @functools.partial(
        pl.kernel,
        out_type=jax.ShapeDtypeStruct((NC * NACC, f), jnp.float32),
        mesh=_mesh(),
        scratch_types=[
            pltpu.VMEM((CH0, L), jnp.int32),
            pltpu.VMEM((8, L), jnp.int32),
            pltpu.VMEM((L, f), jnp.float32),
            pltpu.VMEM_SHARED((NACC, f), jnp.float32),
        ],
    )
    def hist(packed_hbm, out_hbm, packed_v, idxb, ones_v, acc_sh):
        cid = lax.axis_index("c")
        sid = lax.axis_index("s")
        widx = cid * NS + sid
        _fill(ones_v, 0.0)
        _zero_acc(ones_v, acc_sh, sid)
        _fill(ones_v, 1.0)
        pltpu.sync_copy(packed_hbm.at[widx], packed_v.at[pl.ds(0, CHU)])
        plsc.subcore_barrier()

        def body(j, c):
            _unpack_chunk(packed_v, j, idxb, 0, want_src=False)
            pltpu.sync_copy(ones_v, acc_sh.at[idxb.at[1]], add=True)
            return c

        lax.fori_loop(0, CHU, body, 0)
        plsc.subcore_barrier()
        pltpu.sync_copy(
            acc_sh.at[pl.ds(sid * ROWS_PER, ROWS_PER)],
            out_hbm.at[pl.ds(cid * NACC + sid * ROWS_PER, ROWS_PER)],
        )

    return hist


def _make_prop(f):
    @functools.partial(
        pl.kernel,
        out_type=jax.ShapeDtypeStruct((NC * NACC, f), jnp.float32),
        mesh=_mesh(),
        scratch_types=[
            pltpu.VMEM((CH0, L), jnp.int32),
            pltpu.VMEM((8, L), jnp.int32),
            pltpu.VMEM((L, f), jnp.float32),
            pltpu.VMEM((L, f), jnp.float32),
            pltpu.VMEM_SHARED((NACC, f), jnp.float32),
            pltpu.SemaphoreType.DMA,
            pltpu.SemaphoreType.DMA,
        ],
    )
    def prop(hs_hbm, packed_a, packed_b, out_hbm,
             packed_v, idxb, buf0, buf1, acc_sh, sem0, sem1):
        cid = lax.axis_index("c")
        sid = lax.axis_index("s")
        _fill(buf0, 0.0)
        _zero_acc(buf0, acc_sh, sid)

        @pl.when(cid == 0)
        def _sa():
            pltpu.sync_copy(packed_a.at[sid], packed_v)

        @pl.when(cid == 1)
        def _sb():
            pltpu.sync_copy(packed_b.at[sid], packed_v.at[pl.ds(0, CH1)])

        plsc.subcore_barrier()
        nch = jnp.where(cid == 0, CH0, CH1)

        _unpack_chunk(packed_v, 0, idxb, 0, want_src=True)
        _unpack_chunk(packed_v, 1, idxb, 2, want_src=True)
        pltpu.async_copy(hs_hbm.at[idxb.at[0]], buf0, sem0)
        pltpu.async_copy(hs_hbm.at[idxb.at[2]], buf1, sem1)

        def body(i, c):
            j = 2 * i
            pltpu.make_async_copy(hs_hbm.at[idxb.at[0]], buf0, sem0).wait()
            pltpu.sync_copy(buf0, acc_sh.at[idxb.at[1]], add=True)

            @pl.when(j + 2 < nch)
            def _issue0():
                _unpack_chunk(packed_v, j + 2, idxb, 0, want_src=True)
                pltpu.async_copy(hs_hbm.at[idxb.at[0]], buf0, sem0)

            pltpu.make_async_copy(hs_hbm.at[idxb.at[2]], buf1, sem1).wait()
            pltpu.sync_copy(buf1, acc_sh.at[idxb.at[3]], add=True)

            @pl.when(j + 3 < nch)
            def _issue1():
                _unpack_chunk(packed_v, j + 3, idxb, 2, want_src=True)
                pltpu.async_copy(hs_hbm.at[idxb.at[2]], buf1, sem1)

            return c

        lax.fori_loop(0, nch // 2, body, 0)
        plsc.subcore_barrier()
        pltpu.sync_copy(
            acc_sh.at[pl.ds(sid * ROWS_PER, ROWS_PER)],
            out_hbm.at[pl.ds(cid * NACC + sid * ROWS_PER, ROWS_PER)],
        )

    return prop


_BR = 80
_OFF = NACC // _BR


def _tc_first(deg, x, w):
    n, d = x.shape
    h = w.shape[1]

    def body(d0, d1, xr, wr, hs_ref, dinv_ref):
        dg = d0[:, :HW] + d1[:, :HW] + 1.0
        dinv = lax.rsqrt(dg)
        dinv_ref[...] = dinv
        hh = jnp.dot(xr[...], wr[...], preferred_element_type=jnp.float32)
        hs_ref[...] = hh * dinv[:, :1]

    return pl.pallas_call(
        body,
        grid=(n // _BR,),
        in_specs=[
            pl.BlockSpec((_BR, deg.shape[1]), lambda i: (i, 0)),
            pl.BlockSpec((_BR, deg.shape[1]), lambda i: (i + _OFF, 0)),
            pl.BlockSpec((_BR, d), lambda i: (i, 0)),
            pl.BlockSpec((d, h), lambda i: (0, 0)),
        ],
        out_specs=[
            pl.BlockSpec((_BR, h), lambda i: (i, 0)),
            pl.BlockSpec((_BR, HW), lambda i: (i, 0)),
        ],
        out_shape=[
            jax.ShapeDtypeStruct((n, h), jnp.float32),
            jax.ShapeDtypeStruct((n, HW), jnp.float32),
        ],
    )(deg, deg, x, w)


def _tc_mid(p, hs, dinv, b, w):
    n, d = hs.shape
    h = w.shape[1]

    def body(p0r, p1r, hsr, dvr, br, wr, out_ref):
        dv = dvr[:, :1]
        t = jnp.maximum((p0r[...] + p1r[...] + hsr[...]) * dv + br[...], 0.0)
        out_ref[...] = jnp.dot(t, wr[...], preferred_element_type=jnp.float32) * dv

    return pl.pallas_call(
        body,
        grid=(n // _BR,),
        in_specs=[
            pl.BlockSpec((_BR, d), lambda i: (i, 0)),
            pl.BlockSpec((_BR, d), lambda i: (i + _OFF, 0)),
            pl.BlockSpec((_BR, d), lambda i: (i, 0)),
            pl.BlockSpec((_BR, HW), lambda i: (i, 0)),
            pl.BlockSpec((1, d), lambda i: (0, 0)),
            pl.BlockSpec((d, h), lambda i: (0, 0)),
        ],
        out_specs=pl.BlockSpec((_BR, h), lambda i: (i, 0)),
        out_shape=jax.ShapeDtypeStruct((n, h), jnp.float32),
    )(p, p, hs, dinv, b, w)


def _tc_scale(p, hs, dinv, b):
    n, d = hs.shape

    def body(p0r, p1r, hsr, dvr, br, out_ref):
        dv = dvr[:, :1]
        t = jnp.maximum((p0r[...] + p1r[...] + hsr[...]) * dv + br[...], 0.0)
        out_ref[...] = t * dv

    return pl.pallas_call(
        body,
        grid=(n // _BR,),
        in_specs=[
            pl.BlockSpec((_BR, d), lambda i: (i, 0)),
            pl.BlockSpec((_BR, d), lambda i: (i + _OFF, 0)),
            pl.BlockSpec((_BR, d), lambda i: (i, 0)),
            pl.BlockSpec((_BR, HW), lambda i: (i, 0)),
            pl.BlockSpec((1, d), lambda i: (0, 0)),
        ],
        out_specs=pl.BlockSpec((_BR, d), lambda i: (i, 0)),
        out_shape=jax.ShapeDtypeStruct((n, d), jnp.float32),
    )(p, p, hs, dinv, b)


def _tc_final(p, ts, dinv, w, b):
    n, d = ts.shape
    c = w.shape[1]

    def body(p0r, p1r, tsr, dvr, wr, br, out_ref):
        q = jnp.dot(p0r[...] + p1r[...] + tsr[...], wr[...],
                    preferred_element_type=jnp.float32)
        out_ref[...] = q * dvr[:, :1] + br[...]

    return pl.pallas_call(
        body,
        grid=(n // _BR,),
        in_specs=[
            pl.BlockSpec((_BR, d), lambda i: (i, 0)),
            pl.BlockSpec((_BR, d), lambda i: (i + _OFF, 0)),
            pl.BlockSpec((_BR, d), lambda i: (i, 0)),
            pl.BlockSpec((_BR, HW), lambda i: (i, 0)),
            pl.BlockSpec((d, c), lambda i: (0, 0)),
            pl.BlockSpec((1, c), lambda i: (0, 0)),
        ],
        out_specs=pl.BlockSpec((_BR, c), lambda i: (i, 0)),
        out_shape=jax.ShapeDtypeStruct((n, c), jnp.float32),
    )(p, p, ts, dinv, w, b)


def kernel(x, edge_index, batch, W1, b1, W2, b2, W3, b3):
    n, d = x.shape
    e = edge_index.shape[1]
    h = W1.shape[1]

    cap = NW * CHU * L
    pad = cap - e
    packed_flat = jnp.concatenate(
        [(edge_index[0] << 16) | edge_index[1],
         jnp.full((pad,), n, jnp.int32)])
    packed_u = packed_flat.reshape(NW, CHU, L)
    na = NS * CH0 * L
    packed_a = packed_flat[:na].reshape(NS, CH0, L)
    packed_b = packed_flat[na:].reshape(NS, CH1, L)

    deg = _make_hist(h)(packed_u)
    hs1, dinv = _tc_first(deg, x, W1)

    prop_h = _make_prop(h)
    p = prop_h(hs1, packed_a, packed_b)
    hs2 = _tc_mid(p, hs1, dinv, b1.reshape(1, -1), W2)

    p = prop_h(hs2, packed_a, packed_b)
    ts3 = _tc_scale(p, hs2, dinv, b2.reshape(1, -1))

    p3 = prop_h(ts3, packed_a, packed_b)
    return _tc_final(p3, ts3, dinv, W3, b3.reshape(1, -1))


# speedup vs baseline: 2.9283x; 2.9283x over previous
import functools

import jax
import jax.numpy as jnp
from jax import lax
from jax.experimental import pallas as pl
from jax.experimental.pallas import tpu as pltpu
from jax.experimental.pallas import tpu_sc as plsc

NC = 2
NS = 16
NW = NC * NS
L = 128
LANES = 16
HW = 16
NACC = 10240
ROWS_PER = NACC // NS

CH0 = 120
CH1 = 40
CHU = 80
NP = NACC


def _mesh():
    return plsc.VectorSubcoreMesh(
        core_axis_name="c", subcore_axis_name="s", num_cores=NC, num_subcores=NS
    )


def _fill(buf, val):
    f = buf.shape[1]

    def row(r, c):
        for cc in range(f // LANES):
            buf[r, pl.ds(LANES * cc, LANES)] = jnp.full((LANES,), val, jnp.float32)
        return c

    lax.fori_loop(0, buf.shape[0], row, 0)


def _zero_acc(src_v, acc_sh, sid):
    for r5 in range(ROWS_PER // L):
        pltpu.sync_copy(src_v, acc_sh.at[pl.ds(sid * ROWS_PER + L * r5, L)])


def _unpack_chunk(packed_v, j, idxbuf, base, want_src):
    for r in range(L // LANES):
        pv = packed_v[j, pl.ds(LANES * r, LANES)]
        if want_src:
            idxbuf[base, pl.ds(LANES * r, LANES)] = lax.shift_right_logical(pv, 16)
        idxbuf[base + 1, pl.ds(LANES * r, LANES)] = pv & 0xFFFF


def _make_hist(f):
    @functools.partial(
        pl.kernel,
        out_type=jax.ShapeDtypeStruct((NC * NACC, f), jnp.float32),
        mesh=_mesh(),
        scratch_types=[
            pltpu.VMEM((CH0, L), jnp.int32),
            pltpu.VMEM((8, L), jnp.int32),
            pltpu.VMEM((L, f), jnp.float32),
            pltpu.VMEM_SHARED((NACC, f), jnp.float32),
        ],
    )
    def hist(packed_hbm, out_hbm, packed_v, idxb, ones_v, acc_sh):
        cid = lax.axis_index("c")
        sid = lax.axis_index("s")
        widx = cid * NS + sid
        _fill(ones_v, 0.0)
        _zero_acc(ones_v, acc_sh, sid)
        _fill(ones_v, 1.0)
        pltpu.sync_copy(packed_hbm.at[pl.ds(widx * CHU, CHU)],
                        packed_v.at[pl.ds(0, CHU)])
        plsc.subcore_barrier()

        def body(j, c):
            _unpack_chunk(packed_v, j, idxb, 0, want_src=False)
            pltpu.sync_copy(ones_v, acc_sh.at[idxb.at[1]], add=True)
            return c

        lax.fori_loop(0, CHU, body, 0)
        plsc.subcore_barrier()
        pltpu.sync_copy(
            acc_sh.at[pl.ds(sid * ROWS_PER, ROWS_PER)],
            out_hbm.at[pl.ds(cid * NACC + sid * ROWS_PER, ROWS_PER)],
        )

    return hist


def _make_prop(f):
    @functools.partial(
        pl.kernel,
        out_type=jax.ShapeDtypeStruct((NC * NACC, f), jnp.float32),
        mesh=_mesh(),
        scratch_types=[
            pltpu.VMEM((CH0, L), jnp.int32),
            pltpu.VMEM((8, L), jnp.int32),
            pltpu.VMEM((L, f), jnp.float32),
            pltpu.VMEM((L, f), jnp.float32),
            pltpu.VMEM_SHARED((NACC, f), jnp.float32),
            pltpu.SemaphoreType.DMA,
            pltpu.SemaphoreType.DMA,
        ],
    )
    def prop(hs_hbm, packed_hbm, out_hbm,
             packed_v, idxb, buf0, buf1, acc_sh, sem0, sem1):
        cid = lax.axis_index("c")
        sid = lax.axis_index("s")
        _fill(buf0, 0.0)
        _zero_acc(buf0, acc_sh, sid)

        @pl.when(cid == 0)
        def _sa():
            pltpu.sync_copy(packed_hbm.at[pl.ds(sid * CH0, CH0)], packed_v)

        @pl.when(cid == 1)
        def _sb():
            pltpu.sync_copy(packed_hbm.at[pl.ds(NS * CH0 + sid * CH1, CH1)],
                            packed_v.at[pl.ds(0, CH1)])

        plsc.subcore_barrier()
        nch = jnp.where(cid == 0, CH0, CH1)

        _unpack_chunk(packed_v, 0, idxb, 0, want_src=True)
        _unpack_chunk(packed_v, 1, idxb, 2, want_src=True)
        pltpu.async_copy(hs_hbm.at[idxb.at[0]], buf0, sem0)
        pltpu.async_copy(hs_hbm.at[idxb.at[2]], buf1, sem1)

        def body(i, c):
            j = 2 * i
            pltpu.make_async_copy(hs_hbm.at[idxb.at[0]], buf0, sem0).wait()
            pltpu.sync_copy(buf0, acc_sh.at[idxb.at[1]], add=True)

            @pl.when(j + 2 < nch)
            def _issue0():
                _unpack_chunk(packed_v, j + 2, idxb, 0, want_src=True)
                pltpu.async_copy(hs_hbm.at[idxb.at[0]], buf0, sem0)

            pltpu.make_async_copy(hs_hbm.at[idxb.at[2]], buf1, sem1).wait()
            pltpu.sync_copy(buf1, acc_sh.at[idxb.at[3]], add=True)

            @pl.when(j + 3 < nch)
            def _issue1():
                _unpack_chunk(packed_v, j + 3, idxb, 2, want_src=True)
                pltpu.async_copy(hs_hbm.at[idxb.at[2]], buf1, sem1)

            return c

        lax.fori_loop(0, nch // 2, body, 0)
        plsc.subcore_barrier()
        pltpu.sync_copy(
            acc_sh.at[pl.ds(sid * ROWS_PER, ROWS_PER)],
            out_hbm.at[pl.ds(cid * NACC + sid * ROWS_PER, ROWS_PER)],
        )

    return prop


_BR = 1024
_OFF = NACC // _BR


def _tc_first(deg, x, w):
    n, d = x.shape
    h = w.shape[1]

    def body(d0, d1, xr, wr, hs_ref, dinv_ref):
        dg = d0[:, :HW] + d1[:, :HW] + 1.0
        dinv = lax.rsqrt(dg)
        dinv_ref[...] = dinv
        hh = jnp.dot(xr[...], wr[...], preferred_element_type=jnp.float32)
        hs_ref[...] = hh * dinv[:, :1]

    return pl.pallas_call(
        body,
        grid=(n // _BR,),
        in_specs=[
            pl.BlockSpec((_BR, deg.shape[1]), lambda i: (i, 0)),
            pl.BlockSpec((_BR, deg.shape[1]), lambda i: (i + _OFF, 0)),
            pl.BlockSpec((_BR, d), lambda i: (i, 0)),
            pl.BlockSpec((d, h), lambda i: (0, 0)),
        ],
        out_specs=[
            pl.BlockSpec((_BR, h), lambda i: (i, 0)),
            pl.BlockSpec((_BR, HW), lambda i: (i, 0)),
        ],
        out_shape=[
            jax.ShapeDtypeStruct((n, h), jnp.float32),
            jax.ShapeDtypeStruct((n, HW), jnp.float32),
        ],
    )(deg, deg, x, w)


def _tc_mid(p, hs, dinv, b, w):
    n, d = hs.shape
    h = w.shape[1]

    def body(p0r, p1r, hsr, dvr, br, wr, out_ref):
        dv = dvr[:, :1]
        t = jnp.maximum((p0r[...] + p1r[...] + hsr[...]) * dv + br[...], 0.0)
        out_ref[...] = jnp.dot(t, wr[...], preferred_element_type=jnp.float32) * dv

    return pl.pallas_call(
        body,
        grid=(n // _BR,),
        in_specs=[
            pl.BlockSpec((_BR, d), lambda i: (i, 0)),
            pl.BlockSpec((_BR, d), lambda i: (i + _OFF, 0)),
            pl.BlockSpec((_BR, d), lambda i: (i, 0)),
            pl.BlockSpec((_BR, HW), lambda i: (i, 0)),
            pl.BlockSpec((1, d), lambda i: (0, 0)),
            pl.BlockSpec((d, h), lambda i: (0, 0)),
        ],
        out_specs=pl.BlockSpec((_BR, h), lambda i: (i, 0)),
        out_shape=jax.ShapeDtypeStruct((n, h), jnp.float32),
    )(p, p, hs, dinv, b, w)


def _tc_scale(p, hs, dinv, b):
    n, d = hs.shape

    def body(p0r, p1r, hsr, dvr, br, out_ref):
        dv = dvr[:, :1]
        t = jnp.maximum((p0r[...] + p1r[...] + hsr[...]) * dv + br[...], 0.0)
        out_ref[...] = t * dv

    return pl.pallas_call(
        body,
        grid=(n // _BR,),
        in_specs=[
            pl.BlockSpec((_BR, d), lambda i: (i, 0)),
            pl.BlockSpec((_BR, d), lambda i: (i + _OFF, 0)),
            pl.BlockSpec((_BR, d), lambda i: (i, 0)),
            pl.BlockSpec((_BR, HW), lambda i: (i, 0)),
            pl.BlockSpec((1, d), lambda i: (0, 0)),
        ],
        out_specs=pl.BlockSpec((_BR, d), lambda i: (i, 0)),
        out_shape=jax.ShapeDtypeStruct((n, d), jnp.float32),
    )(p, p, hs, dinv, b)


def _tc_final(p, ts, dinv, w, b):
    n, d = ts.shape
    c = w.shape[1]

    def body(p0r, p1r, tsr, dvr, wr, br, out_ref):
        q = jnp.dot(p0r[...] + p1r[...] + tsr[...], wr[...],
                    preferred_element_type=jnp.float32)
        out_ref[...] = q * dvr[:, :1] + br[...]

    return pl.pallas_call(
        body,
        grid=(n // _BR,),
        in_specs=[
            pl.BlockSpec((_BR, d), lambda i: (i, 0)),
            pl.BlockSpec((_BR, d), lambda i: (i + _OFF, 0)),
            pl.BlockSpec((_BR, d), lambda i: (i, 0)),
            pl.BlockSpec((_BR, HW), lambda i: (i, 0)),
            pl.BlockSpec((d, c), lambda i: (0, 0)),
            pl.BlockSpec((1, c), lambda i: (0, 0)),
        ],
        out_specs=pl.BlockSpec((_BR, c), lambda i: (i, 0)),
        out_shape=jax.ShapeDtypeStruct((n, c), jnp.float32),
    )(p, p, ts, dinv, w, b)


def kernel(x, edge_index, batch, W1, b1, W2, b2, W3, b3):
    n, d = x.shape
    e = edge_index.shape[1]
    h = W1.shape[1]

    cap = NW * CHU * L
    pad = cap - e
    pi = jnp.arange(pad, dtype=jnp.int32)
    pad_vals = ((pi & 8191) << 16) | (n + (pi & 127))
    packed = jnp.concatenate(
        [(edge_index[0] << 16) | edge_index[1], pad_vals]).reshape(NW * CHU, L)

    xp = jnp.concatenate([x, jnp.zeros((NP - n, d), x.dtype)])

    deg = _make_hist(h)(packed)
    hs1, dinv = _tc_first(deg, xp, W1)

    prop_h = _make_prop(h)
    p = prop_h(hs1, packed)
    hs2 = _tc_mid(p, hs1, dinv, b1.reshape(1, -1), W2)

    p = prop_h(hs2, packed)
    ts3 = _tc_scale(p, hs2, dinv, b2.reshape(1, -1))

    p3 = prop_h(ts3, packed)
    return _tc_final(p3, ts3, dinv, W3, b3.reshape(1, -1))[:n]


# speedup vs baseline: 3.5430x; 1.2099x over previous
import functools

import jax
import jax.numpy as jnp
from jax import lax
from jax.experimental import pallas as pl
from jax.experimental.pallas import tpu as pltpu
from jax.experimental.pallas import tpu_sc as plsc

NC = 2
NS = 16
NW = NC * NS
L = 128
LANES = 16
HW = 16
NACC = 10240
ROWS_PER = NACC // NS

CH0 = 88
CH1 = 72
CHU = 80
NP = NACC


def _mesh():
    return plsc.VectorSubcoreMesh(
        core_axis_name="c", subcore_axis_name="s", num_cores=NC, num_subcores=NS
    )


def _fill(buf, val):
    f = buf.shape[1]

    def row(r, c):
        for cc in range(f // LANES):
            buf[r, pl.ds(LANES * cc, LANES)] = jnp.full((LANES,), val, jnp.float32)
        return c

    lax.fori_loop(0, buf.shape[0], row, 0)


def _zero_acc(src_v, acc_sh, sid):
    for r5 in range(ROWS_PER // L):
        pltpu.sync_copy(src_v, acc_sh.at[pl.ds(sid * ROWS_PER + L * r5, L)])


def _unpack_chunk(packed_v, j, idxbuf, base, want_src):
    for r in range(L // LANES):
        pv = packed_v[j, pl.ds(LANES * r, LANES)]
        if want_src:
            idxbuf[base, pl.ds(LANES * r, LANES)] = lax.shift_right_logical(pv, 16)
        idxbuf[base + 1, pl.ds(LANES * r, LANES)] = pv & 0xFFFF


def _make_hist(f):
    @functools.partial(
        pl.kernel,
        out_type=jax.ShapeDtypeStruct((NC * NACC, f), jnp.float32),
        mesh=_mesh(),
        scratch_types=[
            pltpu.VMEM((CH0, L), jnp.int32),
            pltpu.VMEM((8, L), jnp.int32),
            pltpu.VMEM((L, f), jnp.float32),
            pltpu.VMEM_SHARED((NACC, f), jnp.float32),
        ],
    )
    def hist(packed_hbm, out_hbm, packed_v, idxb, ones_v, acc_sh):
        cid = lax.axis_index("c")
        sid = lax.axis_index("s")
        widx = cid * NS + sid
        _fill(ones_v, 0.0)
        _zero_acc(ones_v, acc_sh, sid)
        _fill(ones_v, 1.0)
        pltpu.sync_copy(packed_hbm.at[pl.ds(widx * CHU, CHU)],
                        packed_v.at[pl.ds(0, CHU)])
        plsc.subcore_barrier()

        def body(j, c):
            _unpack_chunk(packed_v, j, idxb, 0, want_src=False)
            pltpu.sync_copy(ones_v, acc_sh.at[idxb.at[1]], add=True)
            return c

        lax.fori_loop(0, CHU, body, 0)
        plsc.subcore_barrier()
        pltpu.sync_copy(
            acc_sh.at[pl.ds(sid * ROWS_PER, ROWS_PER)],
            out_hbm.at[pl.ds(cid * NACC + sid * ROWS_PER, ROWS_PER)],
        )

    return hist


def _make_prop(f):
    @functools.partial(
        pl.kernel,
        out_type=jax.ShapeDtypeStruct((NC * NACC, f), jnp.float32),
        mesh=_mesh(),
        scratch_types=[
            pltpu.VMEM((CH0, L), jnp.int32),
            pltpu.VMEM((8, L), jnp.int32),
            pltpu.VMEM((L, f), jnp.float32),
            pltpu.VMEM((L, f), jnp.float32),
            pltpu.VMEM_SHARED((NACC, f), jnp.float32),
            pltpu.SemaphoreType.DMA,
            pltpu.SemaphoreType.DMA,
        ],
    )
    def prop(hs_hbm, packed_hbm, out_hbm,
             packed_v, idxb, buf0, buf1, acc_sh, sem0, sem1):
        cid = lax.axis_index("c")
        sid = lax.axis_index("s")
        _fill(buf0, 0.0)
        _zero_acc(buf0, acc_sh, sid)

        @pl.when(cid == 0)
        def _sa():
            pltpu.sync_copy(packed_hbm.at[pl.ds(sid * CH0, CH0)], packed_v)

        @pl.when(cid == 1)
        def _sb():
            pltpu.sync_copy(packed_hbm.at[pl.ds(NS * CH0 + sid * CH1, CH1)],
                            packed_v.at[pl.ds(0, CH1)])

        plsc.subcore_barrier()
        nch = jnp.where(cid == 0, CH0, CH1)

        _unpack_chunk(packed_v, 0, idxb, 0, want_src=True)
        _unpack_chunk(packed_v, 1, idxb, 2, want_src=True)
        pltpu.async_copy(hs_hbm.at[idxb.at[0]], buf0, sem0)
        pltpu.async_copy(hs_hbm.at[idxb.at[2]], buf1, sem1)

        def body(i, c):
            j = 2 * i
            pltpu.make_async_copy(hs_hbm.at[idxb.at[0]], buf0, sem0).wait()
            pltpu.sync_copy(buf0, acc_sh.at[idxb.at[1]], add=True)

            @pl.when(j + 2 < nch)
            def _issue0():
                _unpack_chunk(packed_v, j + 2, idxb, 0, want_src=True)
                pltpu.async_copy(hs_hbm.at[idxb.at[0]], buf0, sem0)

            pltpu.make_async_copy(hs_hbm.at[idxb.at[2]], buf1, sem1).wait()
            pltpu.sync_copy(buf1, acc_sh.at[idxb.at[3]], add=True)

            @pl.when(j + 3 < nch)
            def _issue1():
                _unpack_chunk(packed_v, j + 3, idxb, 2, want_src=True)
                pltpu.async_copy(hs_hbm.at[idxb.at[2]], buf1, sem1)

            return c

        lax.fori_loop(0, nch // 2, body, 0)
        plsc.subcore_barrier()
        pltpu.sync_copy(
            acc_sh.at[pl.ds(sid * ROWS_PER, ROWS_PER)],
            out_hbm.at[pl.ds(cid * NACC + sid * ROWS_PER, ROWS_PER)],
        )

    return prop


_BR = 1024
_OFF = NACC // _BR


def _tc_first(deg, x, w):
    n, d = x.shape
    h = w.shape[1]

    def body(d0, d1, xr, wr, hs_ref, dinv_ref):
        dg = d0[:, :HW] + d1[:, :HW] + 1.0
        dinv = lax.rsqrt(dg)
        dinv_ref[...] = dinv
        hh = jnp.dot(xr[...], wr[...], preferred_element_type=jnp.float32)
        hs_ref[...] = hh * dinv[:, :1]

    return pl.pallas_call(
        body,
        grid=(n // _BR,),
        in_specs=[
            pl.BlockSpec((_BR, deg.shape[1]), lambda i: (i, 0)),
            pl.BlockSpec((_BR, deg.shape[1]), lambda i: (i + _OFF, 0)),
            pl.BlockSpec((_BR, d), lambda i: (i, 0)),
            pl.BlockSpec((d, h), lambda i: (0, 0)),
        ],
        out_specs=[
            pl.BlockSpec((_BR, h), lambda i: (i, 0)),
            pl.BlockSpec((_BR, HW), lambda i: (i, 0)),
        ],
        out_shape=[
            jax.ShapeDtypeStruct((n, h), jnp.float32),
            jax.ShapeDtypeStruct((n, HW), jnp.float32),
        ],
    )(deg, deg, x, w)


def _tc_mid(p, hs, dinv, b, w):
    n, d = hs.shape
    h = w.shape[1]

    def body(p0r, p1r, hsr, dvr, br, wr, out_ref):
        dv = dvr[:, :1]
        t = jnp.maximum((p0r[...] + p1r[...] + hsr[...]) * dv + br[...], 0.0)
        out_ref[...] = jnp.dot(t, wr[...], preferred_element_type=jnp.float32) * dv

    return pl.pallas_call(
        body,
        grid=(n // _BR,),
        in_specs=[
            pl.BlockSpec((_BR, d), lambda i: (i, 0)),
            pl.BlockSpec((_BR, d), lambda i: (i + _OFF, 0)),
            pl.BlockSpec((_BR, d), lambda i: (i, 0)),
            pl.BlockSpec((_BR, HW), lambda i: (i, 0)),
            pl.BlockSpec((1, d), lambda i: (0, 0)),
            pl.BlockSpec((d, h), lambda i: (0, 0)),
        ],
        out_specs=pl.BlockSpec((_BR, h), lambda i: (i, 0)),
        out_shape=jax.ShapeDtypeStruct((n, h), jnp.float32),
    )(p, p, hs, dinv, b, w)


def _tc_scale(p, hs, dinv, b):
    n, d = hs.shape

    def body(p0r, p1r, hsr, dvr, br, out_ref):
        dv = dvr[:, :1]
        t = jnp.maximum((p0r[...] + p1r[...] + hsr[...]) * dv + br[...], 0.0)
        out_ref[...] = t * dv

    return pl.pallas_call(
        body,
        grid=(n // _BR,),
        in_specs=[
            pl.BlockSpec((_BR, d), lambda i: (i, 0)),
            pl.BlockSpec((_BR, d), lambda i: (i + _OFF, 0)),
            pl.BlockSpec((_BR, d), lambda i: (i, 0)),
            pl.BlockSpec((_BR, HW), lambda i: (i, 0)),
            pl.BlockSpec((1, d), lambda i: (0, 0)),
        ],
        out_specs=pl.BlockSpec((_BR, d), lambda i: (i, 0)),
        out_shape=jax.ShapeDtypeStruct((n, d), jnp.float32),
    )(p, p, hs, dinv, b)


def _tc_final(p, ts, dinv, w, b):
    n, d = ts.shape
    c = w.shape[1]

    def body(p0r, p1r, tsr, dvr, wr, br, out_ref):
        q = jnp.dot(p0r[...] + p1r[...] + tsr[...], wr[...],
                    preferred_element_type=jnp.float32)
        out_ref[...] = q * dvr[:, :1] + br[...]

    return pl.pallas_call(
        body,
        grid=(n // _BR,),
        in_specs=[
            pl.BlockSpec((_BR, d), lambda i: (i, 0)),
            pl.BlockSpec((_BR, d), lambda i: (i + _OFF, 0)),
            pl.BlockSpec((_BR, d), lambda i: (i, 0)),
            pl.BlockSpec((_BR, HW), lambda i: (i, 0)),
            pl.BlockSpec((d, c), lambda i: (0, 0)),
            pl.BlockSpec((1, c), lambda i: (0, 0)),
        ],
        out_specs=pl.BlockSpec((_BR, c), lambda i: (i, 0)),
        out_shape=jax.ShapeDtypeStruct((n, c), jnp.float32),
    )(p, p, ts, dinv, w, b)


def kernel(x, edge_index, batch, W1, b1, W2, b2, W3, b3):
    n, d = x.shape
    e = edge_index.shape[1]
    h = W1.shape[1]

    cap = NW * CHU * L
    pad = cap - e
    pi = jnp.arange(pad, dtype=jnp.int32)
    pad_vals = ((pi & 8191) << 16) | (n + (pi & 127))
    packed = jnp.concatenate(
        [(edge_index[0] << 16) | edge_index[1], pad_vals]).reshape(NW * CHU, L)

    xp = jnp.concatenate([x, jnp.zeros((NP - n, d), x.dtype)])

    deg = _make_hist(h)(packed)
    hs1, dinv = _tc_first(deg, xp, W1)

    prop_h = _make_prop(h)
    p = prop_h(hs1, packed)
    hs2 = _tc_mid(p, hs1, dinv, b1.reshape(1, -1), W2)

    p = prop_h(hs2, packed)
    ts3 = _tc_scale(p, hs2, dinv, b2.reshape(1, -1))

    p3 = prop_h(ts3, packed)
    return _tc_final(p3, ts3, dinv, W3, b3.reshape(1, -1))[:n]
